# baseline (speedup 1.0000x reference)
import functools

import jax
import jax.numpy as jnp
from jax import lax
from jax.experimental import pallas as pl
from jax.experimental.pallas import tpu as pltpu
from jax.experimental.pallas import tpu_sc as plsc

RAP = 0.1
PRIOR_EPS = 1e-4
PROB_EPS = 1e-12

E = 1048576
B = 4096
NC, NS, L = 2, 16, 16
NW = NC * NS
C_PER = E // NW
ROWS_PER = C_PER // 128
S1_SUB_ROWS = 64
S2_SUB = 16384
_mesh = plsc.VectorSubcoreMesh(core_axis_name="c", subcore_axis_name="s")
_sc_params = pltpu.CompilerParams(needs_layout_passes=False)


def _e1_body(el_ref, es_ref, u_ref, p_ref, w_ref):
    p_ref[...] = jnp.exp(el_ref[...]) * jnp.sqrt(
        jnp.clip(es_ref[...], PRIOR_EPS, None))
    uc = jnp.clip(u_ref[...], 1e-9, 1.0 - 1e-9)
    w_ref[...] = -1.0 / jnp.log(uc)


def _e1(edge_logits, edge_scores, u):
    rows = E // 128
    grid = 16
    blk = rows // grid
    spec = pl.BlockSpec((blk, 128), lambda i: (i, 0))
    return pl.pallas_call(
        _e1_body,
        grid=(grid,),
        in_specs=[spec] * 3,
        out_specs=[spec] * 2,
        out_shape=[jax.ShapeDtypeStruct((rows, 128), jnp.float32)] * 2,
    )(edge_logits.reshape(rows, 128), edge_scores.reshape(rows, 128),
      u.reshape(rows, 128))


def _fill(ref, n, value):
    v = jnp.full((L,), value, dtype=ref.dtype)

    def body(i, _):
        ref[pl.ds(i * L, L)] = v
        return 0

    lax.fori_loop(0, n // L, body, 0)


SL = B // NS


@functools.partial(
    pl.kernel,
    mesh=_mesh,
    out_type=[jax.ShapeDtypeStruct((NC, B), jnp.float32),
              jax.ShapeDtypeStruct((NC, B), jnp.float32)],
    scratch_types=[
        pltpu.VMEM((C_PER,), jnp.int32),
        pltpu.VMEM((C_PER,), jnp.float32),
        pltpu.VMEM((B,), jnp.float32),
        pltpu.VMEM((B,), jnp.float32),
        pltpu.VMEM((NS, SL), jnp.float32),
        pltpu.VMEM((NS, SL), jnp.float32),
        pltpu.VMEM((SL,), jnp.float32),
        pltpu.VMEM((SL,), jnp.float32),
        pltpu.VMEM_SHARED((NS, NS, SL), jnp.float32),
        pltpu.VMEM_SHARED((NS, NS, SL), jnp.float32),
        pltpu.SemaphoreType.DMA,
        pltpu.SemaphoreType.DMA,
    ],
    compiler_params=_sc_params,
)
def _s1(ids1d, p1d, zpart, cpart, ids_s, p_s, zloc, cloc, mz, mc,
        zred, cred, zsl, csl, sem0, sem1):
    c = lax.axis_index("c")
    s = lax.axis_index("s")
    wid = s * NC + c
    base = wid * C_PER

    cp0 = pltpu.async_copy(ids1d.at[pl.ds(base, C_PER)], ids_s, sem0)
    cp1 = pltpu.async_copy(p1d.at[pl.ds(base, C_PER)], p_s, sem1)
    _fill(zloc, B, 0.0)
    _fill(cloc, B, 0.0)
    cp0.wait()
    cp1.wait()

    iota = lax.broadcasted_iota(jnp.int32, (L,), 0)
    nxt_idx = jnp.minimum(iota + 1, L - 1)
    ones = jnp.ones((L,), jnp.float32)

    def one_vreg(v):
        ds = pl.ds(v * L, L)
        ids16 = ids_s[ds]
        sp = p_s[ds]
        sc = ones
        for d in (1, 2, 4, 8):
            idxs = jnp.maximum(iota - d, 0)
            sh_id = ids16.at[idxs].get(mode="promise_in_bounds")
            sh_p = sp.at[idxs].get(mode="promise_in_bounds")
            sh_c = sc.at[idxs].get(mode="promise_in_bounds")
            ok = (sh_id == ids16) & (iota >= d)
            sp = jnp.where(ok, sp + sh_p, sp)
            sc = jnp.where(ok, sc + sh_c, sc)
        nxt_id = ids16.at[nxt_idx].get(mode="promise_in_bounds")
        is_last = (ids16 != nxt_id) | (iota == L - 1)
        plsc.addupdate_scatter(zloc, [ids16], sp, mask=is_last)
        plsc.addupdate_scatter(cloc, [ids16], sc, mask=is_last)

    def body(v2, _):
        one_vreg(2 * v2)
        one_vreg(2 * v2 + 1)
        return 0

    lax.fori_loop(0, C_PER // (2 * L), body, 0)

    for w in range(NS):
        pltpu.sync_copy(zloc.at[pl.ds(w * SL, SL)], zsl.at[w, s])
        pltpu.sync_copy(cloc.at[pl.ds(w * SL, SL)], csl.at[w, s])
    plsc.subcore_barrier()
    pltpu.sync_copy(zsl.at[s], mz)
    pltpu.sync_copy(csl.at[s], mc)

    def red(i, _):
        ds = pl.ds(i * L, L)
        az = mz[0, ds]
        ac = mc[0, ds]
        for r in range(1, NS):
            az = az + mz[r, ds]
            ac = ac + mc[r, ds]
        zred[ds] = az
        cred[ds] = ac
        return 0

    lax.fori_loop(0, SL // L, red, 0)
    pltpu.sync_copy(zred, zpart.at[c, pl.ds(s * SL, SL)])
    pltpu.sync_copy(cred, cpart.at[c, pl.ds(s * SL, SL)])


N_SUB = C_PER // S2_SUB


@functools.partial(
    pl.kernel,
    mesh=_mesh,
    out_type=[jax.ShapeDtypeStruct((NW, B), jnp.float32),
              jax.ShapeDtypeStruct((NW, B), jnp.float32)],
    scratch_types=[
        pltpu.VMEM((B,), jnp.float32),
        pltpu.VMEM((B,), jnp.float32),
        pltpu.VMEM((B,), jnp.float32),
        pltpu.VMEM((B,), jnp.float32),
        pltpu.VMEM((C_PER,), jnp.int32),
        pltpu.VMEM((C_PER,), jnp.float32),
        pltpu.VMEM((C_PER,), jnp.float32),
        pltpu.SemaphoreType.DMA,
        pltpu.SemaphoreType.DMA,
        pltpu.SemaphoreType.DMA,
    ],
    compiler_params=_sc_params,
)
def _s2(ids1d, p1d, w1d, zpart, cpart, stop, bestv, bestq,
        dinv_s, it_s, mv_s, mq_s, ids_s, p_s, w_s, sem0, sem1, sem2):
    c = lax.axis_index("c")
    s = lax.axis_index("s")
    wid = s * NC + c

    sems = (sem0, sem1)
    for sub in range(N_SUB):
        base = wid * C_PER + sub * S2_SUB
        dst = pl.ds(sub * S2_SUB, S2_SUB)
        pltpu.async_copy(ids1d.at[pl.ds(base, S2_SUB)], ids_s.at[dst],
                         sems[sub])
        pltpu.async_copy(p1d.at[pl.ds(base, S2_SUB)], p_s.at[dst], sems[sub])
        pltpu.async_copy(w1d.at[pl.ds(base, S2_SUB)], w_s.at[dst], sems[sub])

    pltpu.sync_copy(zpart.at[0], mv_s)
    pltpu.sync_copy(zpart.at[1], mq_s)

    def pro_z(i, _):
        ds = pl.ds(i * L, L)
        dinv_s[ds] = mv_s[ds] + mq_s[ds]
        return 0

    lax.fori_loop(0, B // L, pro_z, 0)
    pltpu.sync_copy(cpart.at[0], mv_s)
    pltpu.sync_copy(cpart.at[1], mq_s)

    def pro_c(i, _):
        ds = pl.ds(i * L, L)
        it_s[ds] = 1.0 / (mv_s[ds] + mq_s[ds] + 1.0)
        return 0

    lax.fori_loop(0, B // L, pro_c, 0)
    pltpu.sync_copy(stop, mv_s)

    def pro_d(i, _):
        ds = pl.ds(i * L, L)
        dinv_s[ds] = 1.0 / (dinv_s[ds] + jnp.exp(mv_s[ds]))
        zero = jnp.zeros((L,), jnp.float32)
        mv_s[ds] = zero
        mq_s[ds] = zero
        return 0

    lax.fori_loop(0, B // L, pro_d, 0)

    iota = lax.broadcasted_iota(jnp.int32, (L,), 0)
    nxt_idx = jnp.minimum(iota + 1, L - 1)

    def one_vreg(ids_ref, p_ref, w_ref, v):
        ds = pl.ds(v * L, L)
        ids16 = ids_ref[ds]
        pv = p_ref[ds]
        wv = w_ref[ds]
        dg = plsc.load_gather(dinv_s, [ids16])
        ig = plsc.load_gather(it_s, [ids16])
        q = (1.0 - RAP) * pv * dg + RAP * ig
        cur_s = q * wv
        cur_q = q
        for d in (1, 2, 4, 8):
            idxs = jnp.maximum(iota - d, 0)
            sh_s = cur_s.at[idxs].get(mode="promise_in_bounds")
            sh_q = cur_q.at[idxs].get(mode="promise_in_bounds")
            sh_id = ids16.at[idxs].get(mode="promise_in_bounds")
            same = sh_id == ids16
            cur_q = jnp.where(same & (sh_s >= cur_s), sh_q, cur_q)
            cur_s = jnp.where(same, jnp.maximum(sh_s, cur_s), cur_s)
        nxt_id = ids16.at[nxt_idx].get(mode="promise_in_bounds")
        is_last = (ids16 != nxt_id) | (iota == L - 1)
        mv = plsc.load_gather(mv_s, [ids16])
        upd = is_last & (cur_s > mv)
        plsc.store_scatter(mv_s, [ids16], cur_s, mask=upd)
        plsc.store_scatter(mq_s, [ids16], cur_q, mask=upd)

    for sub in range(N_SUB):
        dst = pl.ds(sub * S2_SUB, S2_SUB)
        for _ in range(3):
            pltpu.make_async_copy(
                ids1d.at[pl.ds(0, S2_SUB)], ids_s.at[dst], sems[sub]).wait()

        off = sub * S2_SUB // L

        def body(v2, _):
            one_vreg(ids_s, p_s, w_s, off + 2 * v2)
            one_vreg(ids_s, p_s, w_s, off + 2 * v2 + 1)
            return 0

        lax.fori_loop(0, S2_SUB // (2 * L), body, 0)

    pltpu.async_copy(mv_s, bestv.at[wid], sem2)
    pltpu.async_copy(mq_s, bestq.at[wid], sem2)
    pltpu.make_async_copy(mv_s, bestv.at[wid], sem2).wait()
    pltpu.make_async_copy(mq_s, bestq.at[wid], sem2).wait()


def _e3_body(bestv_ref, bestq_ref, zpart_ref, cpart_ref, stop_ref, out_ref):
    best = bestv_ref[0]
    q = bestq_ref[0]
    for w in range(1, NW):
        v = bestv_ref[w]
        upd = v > best
        best = jnp.where(upd, v, best)
        q = jnp.where(upd, bestq_ref[w], q)
    z = zpart_ref[0] + zpart_ref[1]
    cnt = cpart_ref[0] + cpart_ref[1]
    exp_stop = jnp.exp(stop_ref[...])
    dinv = 1.0 / (z + exp_stop)
    it = 1.0 / (cnt + 1.0)
    final_stop = (1.0 - RAP) * exp_stop * dinv + RAP * it
    log_stop = jnp.log(jnp.clip(final_stop, PROB_EPS, None))
    log_edge = jnp.log(jnp.clip(q, PROB_EPS, None))
    out_ref[...] = jnp.where(final_stop >= best, log_stop, log_edge)


def _e3(bestv, bestq, zpart, cpart, stop_logits):
    rb = B // 128
    out = pl.pallas_call(
        _e3_body,
        out_shape=jax.ShapeDtypeStruct((rb, 128), jnp.float32),
    )(bestv.reshape(NW, rb, 128), bestq.reshape(NW, rb, 128),
      zpart.reshape(NC, rb, 128), cpart.reshape(NC, rb, 128),
      stop_logits.reshape(rb, 128))
    return out.reshape(B)


def kernel(edge_logits, stop_logits, edge_scores, u, edge_batch):
    p2, w2 = _e1(edge_logits, edge_scores, u)
    zpart, cpart = _s1(edge_batch, p2.reshape(E))
    bestv, bestq = _s2(edge_batch, p2.reshape(E), w2.reshape(E),
                       zpart, cpart, stop_logits)
    return _e3(bestv, bestq, zpart, cpart, stop_logits)


# speedup vs baseline: 1.1406x; 1.1406x over previous
import functools

import jax
import jax.numpy as jnp
from jax import lax
from jax.experimental import pallas as pl
from jax.experimental.pallas import tpu as pltpu
from jax.experimental.pallas import tpu_sc as plsc

RAP = 0.1
PRIOR_EPS = 1e-4
PROB_EPS = 1e-12

E = 1048576
B = 4096
NC, NS, L = 2, 16, 16
NW = NC * NS
C_PER = E // NW
ROWS_PER = C_PER // 128
S1_SUB_ROWS = 64
S2_SUB = 16384
_mesh = plsc.VectorSubcoreMesh(core_axis_name="c", subcore_axis_name="s")
_sc_params = pltpu.CompilerParams(needs_layout_passes=False)


def _e1_body(el_ref, es_ref, u_ref, p_ref, w_ref):
    p_ref[...] = jnp.exp(el_ref[...]) * jnp.sqrt(
        jnp.clip(es_ref[...], PRIOR_EPS, None))
    uc = jnp.clip(u_ref[...], 1e-9, 1.0 - 1e-9)
    w_ref[...] = -1.0 / jnp.log(uc)


def _e1(edge_logits, edge_scores, u):
    rows = E // 128
    grid = 16
    blk = rows // grid
    spec = pl.BlockSpec((blk, 128), lambda i: (i, 0))
    return pl.pallas_call(
        _e1_body,
        grid=(grid,),
        in_specs=[spec] * 3,
        out_specs=[spec] * 2,
        out_shape=[jax.ShapeDtypeStruct((rows, 128), jnp.float32)] * 2,
    )(edge_logits.reshape(rows, 128), edge_scores.reshape(rows, 128),
      u.reshape(rows, 128))


def _fill(ref, n, value):
    v = jnp.full((L,), value, dtype=ref.dtype)

    def body(i, _):
        ref[pl.ds(i * L, L)] = v
        return 0

    lax.fori_loop(0, n // L, body, 0)


SL = B // NS


@functools.partial(
    pl.kernel,
    mesh=_mesh,
    out_type=[jax.ShapeDtypeStruct((NC, B), jnp.float32),
              jax.ShapeDtypeStruct((NC, B), jnp.float32)],
    scratch_types=[
        pltpu.VMEM((C_PER,), jnp.int32),
        pltpu.VMEM((C_PER,), jnp.float32),
        pltpu.VMEM((B,), jnp.float32),
        pltpu.VMEM((B,), jnp.float32),
        pltpu.VMEM((NS, SL), jnp.float32),
        pltpu.VMEM((NS, SL), jnp.float32),
        pltpu.VMEM((SL,), jnp.float32),
        pltpu.VMEM((SL,), jnp.float32),
        pltpu.VMEM_SHARED((NS, NS, SL), jnp.float32),
        pltpu.VMEM_SHARED((NS, NS, SL), jnp.float32),
        pltpu.SemaphoreType.DMA,
        pltpu.SemaphoreType.DMA,
    ],
    compiler_params=_sc_params,
)
def _s1(ids1d, p1d, zpart, cpart, ids_s, p_s, zloc, cloc, mz, mc,
        zred, cred, zsl, csl, sem0, sem1):
    c = lax.axis_index("c")
    s = lax.axis_index("s")
    wid = s * NC + c
    base = wid * C_PER

    cp0 = pltpu.async_copy(ids1d.at[pl.ds(base, C_PER)], ids_s, sem0)
    cp1 = pltpu.async_copy(p1d.at[pl.ds(base, C_PER)], p_s, sem1)
    _fill(zloc, B, 0.0)
    _fill(cloc, B, 0.0)
    cp0.wait()
    cp1.wait()

    iota = lax.broadcasted_iota(jnp.int32, (L,), 0)
    nxt_idx = jnp.minimum(iota + 1, L - 1)
    ones = jnp.ones((L,), jnp.float32)

    def one_vreg(v):
        ds = pl.ds(v * L, L)
        ids16 = ids_s[ds]
        sp = p_s[ds]
        sc = ones
        for d in (1, 2, 4, 8):
            idxs = jnp.maximum(iota - d, 0)
            sh_id = ids16.at[idxs].get(mode="promise_in_bounds")
            sh_p = sp.at[idxs].get(mode="promise_in_bounds")
            sh_c = sc.at[idxs].get(mode="promise_in_bounds")
            ok = (sh_id == ids16) & (iota >= d)
            sp = jnp.where(ok, sp + sh_p, sp)
            sc = jnp.where(ok, sc + sh_c, sc)
        nxt_id = ids16.at[nxt_idx].get(mode="promise_in_bounds")
        is_last = (ids16 != nxt_id) | (iota == L - 1)
        plsc.addupdate_scatter(zloc, [ids16], sp, mask=is_last)
        plsc.addupdate_scatter(cloc, [ids16], sc, mask=is_last)

    @plsc.parallel_loop(0, C_PER // L, unroll=4)
    def _(v):
        one_vreg(v)

    for w in range(NS):
        pltpu.sync_copy(zloc.at[pl.ds(w * SL, SL)], zsl.at[w, s])
        pltpu.sync_copy(cloc.at[pl.ds(w * SL, SL)], csl.at[w, s])
    plsc.subcore_barrier()
    pltpu.sync_copy(zsl.at[s], mz)
    pltpu.sync_copy(csl.at[s], mc)

    def red(i, _):
        ds = pl.ds(i * L, L)
        az = mz[0, ds]
        ac = mc[0, ds]
        for r in range(1, NS):
            az = az + mz[r, ds]
            ac = ac + mc[r, ds]
        zred[ds] = az
        cred[ds] = ac
        return 0

    lax.fori_loop(0, SL // L, red, 0)
    pltpu.sync_copy(zred, zpart.at[c, pl.ds(s * SL, SL)])
    pltpu.sync_copy(cred, cpart.at[c, pl.ds(s * SL, SL)])


N_SUB = C_PER // S2_SUB


@functools.partial(
    pl.kernel,
    mesh=_mesh,
    out_type=[jax.ShapeDtypeStruct((NW, B), jnp.float32),
              jax.ShapeDtypeStruct((NW, B), jnp.float32)],
    scratch_types=[
        pltpu.VMEM((B,), jnp.float32),
        pltpu.VMEM((B,), jnp.float32),
        pltpu.VMEM((B,), jnp.float32),
        pltpu.VMEM((B,), jnp.float32),
        pltpu.VMEM((C_PER,), jnp.int32),
        pltpu.VMEM((C_PER,), jnp.float32),
        pltpu.VMEM((C_PER,), jnp.float32),
        pltpu.SemaphoreType.DMA,
        pltpu.SemaphoreType.DMA,
        pltpu.SemaphoreType.DMA,
    ],
    compiler_params=_sc_params,
)
def _s2(ids1d, p1d, w1d, zpart, cpart, stop, bestv, bestq,
        dinv_s, it_s, mv_s, mq_s, ids_s, p_s, w_s, sem0, sem1, sem2):
    c = lax.axis_index("c")
    s = lax.axis_index("s")
    wid = s * NC + c

    sems = (sem0, sem1)
    for sub in range(N_SUB):
        base = wid * C_PER + sub * S2_SUB
        dst = pl.ds(sub * S2_SUB, S2_SUB)
        pltpu.async_copy(ids1d.at[pl.ds(base, S2_SUB)], ids_s.at[dst],
                         sems[sub])
        pltpu.async_copy(p1d.at[pl.ds(base, S2_SUB)], p_s.at[dst], sems[sub])
        pltpu.async_copy(w1d.at[pl.ds(base, S2_SUB)], w_s.at[dst], sems[sub])

    pltpu.sync_copy(zpart.at[0], mv_s)
    pltpu.sync_copy(zpart.at[1], mq_s)

    def pro_z(i, _):
        ds = pl.ds(i * L, L)
        dinv_s[ds] = mv_s[ds] + mq_s[ds]
        return 0

    lax.fori_loop(0, B // L, pro_z, 0)
    pltpu.sync_copy(cpart.at[0], mv_s)
    pltpu.sync_copy(cpart.at[1], mq_s)

    def pro_c(i, _):
        ds = pl.ds(i * L, L)
        it_s[ds] = 1.0 / (mv_s[ds] + mq_s[ds] + 1.0)
        return 0

    lax.fori_loop(0, B // L, pro_c, 0)
    pltpu.sync_copy(stop, mv_s)

    def pro_d(i, _):
        ds = pl.ds(i * L, L)
        dinv_s[ds] = 1.0 / (dinv_s[ds] + jnp.exp(mv_s[ds]))
        zero = jnp.zeros((L,), jnp.float32)
        mv_s[ds] = zero
        mq_s[ds] = zero
        return 0

    lax.fori_loop(0, B // L, pro_d, 0)

    iota = lax.broadcasted_iota(jnp.int32, (L,), 0)
    nxt_idx = jnp.minimum(iota + 1, L - 1)

    def one_vreg(ids_ref, p_ref, w_ref, v):
        ds = pl.ds(v * L, L)
        ids16 = ids_ref[ds]
        pv = p_ref[ds]
        wv = w_ref[ds]
        dg = plsc.load_gather(dinv_s, [ids16])
        ig = plsc.load_gather(it_s, [ids16])
        q = (1.0 - RAP) * pv * dg + RAP * ig
        cur_s = q * wv
        cur_q = q
        for d in (1, 2, 4, 8):
            idxs = jnp.maximum(iota - d, 0)
            sh_s = cur_s.at[idxs].get(mode="promise_in_bounds")
            sh_q = cur_q.at[idxs].get(mode="promise_in_bounds")
            sh_id = ids16.at[idxs].get(mode="promise_in_bounds")
            same = sh_id == ids16
            cur_q = jnp.where(same & (sh_s >= cur_s), sh_q, cur_q)
            cur_s = jnp.where(same, jnp.maximum(sh_s, cur_s), cur_s)
        nxt_id = ids16.at[nxt_idx].get(mode="promise_in_bounds")
        is_last = (ids16 != nxt_id) | (iota == L - 1)
        mv = plsc.load_gather(mv_s, [ids16])
        upd = is_last & (cur_s > mv)
        plsc.store_scatter(mv_s, [ids16], cur_s, mask=upd)
        plsc.store_scatter(mq_s, [ids16], cur_q, mask=upd)

    for sub in range(N_SUB):
        dst = pl.ds(sub * S2_SUB, S2_SUB)
        for _ in range(3):
            pltpu.make_async_copy(
                ids1d.at[pl.ds(0, S2_SUB)], ids_s.at[dst], sems[sub]).wait()

        off = sub * S2_SUB // L

        def body(v2, _):
            one_vreg(ids_s, p_s, w_s, off + 2 * v2)
            one_vreg(ids_s, p_s, w_s, off + 2 * v2 + 1)
            return 0

        lax.fori_loop(0, S2_SUB // (2 * L), body, 0)

    pltpu.async_copy(mv_s, bestv.at[wid], sem2)
    pltpu.async_copy(mq_s, bestq.at[wid], sem2)
    pltpu.make_async_copy(mv_s, bestv.at[wid], sem2).wait()
    pltpu.make_async_copy(mq_s, bestq.at[wid], sem2).wait()


def _e3_body(bestv_ref, bestq_ref, zpart_ref, cpart_ref, stop_ref, out_ref):
    best = bestv_ref[0]
    q = bestq_ref[0]
    for w in range(1, NW):
        v = bestv_ref[w]
        upd = v > best
        best = jnp.where(upd, v, best)
        q = jnp.where(upd, bestq_ref[w], q)
    z = zpart_ref[0] + zpart_ref[1]
    cnt = cpart_ref[0] + cpart_ref[1]
    exp_stop = jnp.exp(stop_ref[...])
    dinv = 1.0 / (z + exp_stop)
    it = 1.0 / (cnt + 1.0)
    final_stop = (1.0 - RAP) * exp_stop * dinv + RAP * it
    log_stop = jnp.log(jnp.clip(final_stop, PROB_EPS, None))
    log_edge = jnp.log(jnp.clip(q, PROB_EPS, None))
    out_ref[...] = jnp.where(final_stop >= best, log_stop, log_edge)


def _e3(bestv, bestq, zpart, cpart, stop_logits):
    rb = B // 128
    out = pl.pallas_call(
        _e3_body,
        out_shape=jax.ShapeDtypeStruct((rb, 128), jnp.float32),
    )(bestv.reshape(NW, rb, 128), bestq.reshape(NW, rb, 128),
      zpart.reshape(NC, rb, 128), cpart.reshape(NC, rb, 128),
      stop_logits.reshape(rb, 128))
    return out.reshape(B)


def kernel(edge_logits, stop_logits, edge_scores, u, edge_batch):
    p2, w2 = _e1(edge_logits, edge_scores, u)
    zpart, cpart = _s1(edge_batch, p2.reshape(E))
    bestv, bestq = _s2(edge_batch, p2.reshape(E), w2.reshape(E),
                       zpart, cpart, stop_logits)
    return _e3(bestv, bestq, zpart, cpart, stop_logits)


# speedup vs baseline: 1.2903x; 1.1313x over previous
import functools

import jax
import jax.numpy as jnp
from jax import lax
from jax.experimental import pallas as pl
from jax.experimental.pallas import tpu as pltpu
from jax.experimental.pallas import tpu_sc as plsc

RAP = 0.1
PRIOR_EPS = 1e-4
PROB_EPS = 1e-12

E = 1048576
B = 4096
NC, NS, L = 2, 16, 16
NW = NC * NS
C_PER = E // NW
ROWS_PER = C_PER // 128
S1_SUB_ROWS = 64
S2_SUB = 16384
_mesh = plsc.VectorSubcoreMesh(core_axis_name="c", subcore_axis_name="s")
_sc_params = pltpu.CompilerParams(needs_layout_passes=False)


def _e1_body(el_ref, es_ref, u_ref, p_ref, w_ref):
    p_ref[...] = jnp.exp(el_ref[...]) * jnp.sqrt(
        jnp.clip(es_ref[...], PRIOR_EPS, None))
    uc = jnp.clip(u_ref[...], 1e-9, 1.0 - 1e-9)
    w_ref[...] = -1.0 / jnp.log(uc)


def _e1(edge_logits, edge_scores, u):
    rows = E // 128
    grid = 16
    blk = rows // grid
    spec = pl.BlockSpec((blk, 128), lambda i: (i, 0))
    return pl.pallas_call(
        _e1_body,
        grid=(grid,),
        in_specs=[spec] * 3,
        out_specs=[spec] * 2,
        out_shape=[jax.ShapeDtypeStruct((rows, 128), jnp.float32)] * 2,
    )(edge_logits.reshape(rows, 128), edge_scores.reshape(rows, 128),
      u.reshape(rows, 128))


def _fill(ref, n, value):
    v = jnp.full((L,), value, dtype=ref.dtype)

    def body(i, _):
        ref[pl.ds(i * L, L)] = v
        return 0

    lax.fori_loop(0, n // L, body, 0)


SL = B // NS


@functools.partial(
    pl.kernel,
    mesh=_mesh,
    out_type=[jax.ShapeDtypeStruct((NC, B), jnp.float32),
              jax.ShapeDtypeStruct((NC, B), jnp.float32)],
    scratch_types=[
        pltpu.VMEM((C_PER,), jnp.int32),
        pltpu.VMEM((C_PER,), jnp.float32),
        pltpu.VMEM((B,), jnp.float32),
        pltpu.VMEM((B,), jnp.float32),
        pltpu.VMEM((NS, SL), jnp.float32),
        pltpu.VMEM((NS, SL), jnp.float32),
        pltpu.VMEM((SL,), jnp.float32),
        pltpu.VMEM((SL,), jnp.float32),
        pltpu.VMEM_SHARED((NS, NS, SL), jnp.float32),
        pltpu.VMEM_SHARED((NS, NS, SL), jnp.float32),
        pltpu.SemaphoreType.DMA,
        pltpu.SemaphoreType.DMA,
    ],
    compiler_params=_sc_params,
)
def _s1(ids1d, p1d, zpart, cpart, ids_s, p_s, zloc, cloc, mz, mc,
        zred, cred, zsl, csl, sem0, sem1):
    c = lax.axis_index("c")
    s = lax.axis_index("s")
    wid = s * NC + c
    base = wid * C_PER

    cp0 = pltpu.async_copy(ids1d.at[pl.ds(base, C_PER)], ids_s, sem0)
    cp1 = pltpu.async_copy(p1d.at[pl.ds(base, C_PER)], p_s, sem1)
    _fill(zloc, B, 0.0)
    _fill(cloc, B, 0.0)
    cp0.wait()
    cp1.wait()

    iota = lax.broadcasted_iota(jnp.int32, (L,), 0)
    nxt_idx = jnp.minimum(iota + 1, L - 1)
    ones = jnp.ones((L,), jnp.float32)

    def one_vreg(v):
        ds = pl.ds(v * L, L)
        ids16 = ids_s[ds]
        sp = p_s[ds]
        sc = ones
        for d in (1, 2, 4, 8):
            idxs = jnp.maximum(iota - d, 0)
            sh_id = ids16.at[idxs].get(mode="promise_in_bounds")
            sh_p = sp.at[idxs].get(mode="promise_in_bounds")
            sh_c = sc.at[idxs].get(mode="promise_in_bounds")
            ok = (sh_id == ids16) & (iota >= d)
            sp = jnp.where(ok, sp + sh_p, sp)
            sc = jnp.where(ok, sc + sh_c, sc)
        nxt_id = ids16.at[nxt_idx].get(mode="promise_in_bounds")
        is_last = (ids16 != nxt_id) | (iota == L - 1)
        plsc.addupdate_scatter(zloc, [ids16], sp, mask=is_last)
        plsc.addupdate_scatter(cloc, [ids16], sc, mask=is_last)

    @plsc.parallel_loop(0, C_PER // L, unroll=4)
    def _(v):
        one_vreg(v)

    for w in range(NS):
        pltpu.sync_copy(zloc.at[pl.ds(w * SL, SL)], zsl.at[w, s])
        pltpu.sync_copy(cloc.at[pl.ds(w * SL, SL)], csl.at[w, s])
    plsc.subcore_barrier()
    pltpu.sync_copy(zsl.at[s], mz)
    pltpu.sync_copy(csl.at[s], mc)

    def red(i, _):
        ds = pl.ds(i * L, L)
        az = mz[0, ds]
        ac = mc[0, ds]
        for r in range(1, NS):
            az = az + mz[r, ds]
            ac = ac + mc[r, ds]
        zred[ds] = az
        cred[ds] = ac
        return 0

    lax.fori_loop(0, SL // L, red, 0)
    pltpu.sync_copy(zred, zpart.at[c, pl.ds(s * SL, SL)])
    pltpu.sync_copy(cred, cpart.at[c, pl.ds(s * SL, SL)])


N_SUB = C_PER // S2_SUB


@functools.partial(
    pl.kernel,
    mesh=_mesh,
    out_type=[jax.ShapeDtypeStruct((NW, B), jnp.float32),
              jax.ShapeDtypeStruct((NW, B), jnp.float32)],
    scratch_types=[
        pltpu.VMEM((B,), jnp.float32),
        pltpu.VMEM((B,), jnp.float32),
        pltpu.VMEM((B,), jnp.float32),
        pltpu.VMEM((B,), jnp.float32),
        pltpu.VMEM((C_PER,), jnp.int32),
        pltpu.VMEM((C_PER,), jnp.float32),
        pltpu.VMEM((C_PER,), jnp.float32),
        pltpu.VMEM((2 * C_PER // L,), jnp.int32),
        pltpu.VMEM((2 * C_PER // L,), jnp.float32),
        pltpu.VMEM((2 * C_PER // L,), jnp.float32),
        pltpu.SemaphoreType.DMA,
        pltpu.SemaphoreType.DMA,
        pltpu.SemaphoreType.DMA,
    ],
    compiler_params=_sc_params,
)
def _s2(ids1d, p1d, w1d, zpart, cpart, stop, bestv, bestq,
        dinv_s, it_s, mv_s, mq_s, ids_s, p_s, w_s, ei_s, es_s, eq_s,
        sem0, sem1, sem2):
    c = lax.axis_index("c")
    s = lax.axis_index("s")
    wid = s * NC + c

    sems = (sem0, sem1)
    for sub in range(N_SUB):
        base = wid * C_PER + sub * S2_SUB
        dst = pl.ds(sub * S2_SUB, S2_SUB)
        pltpu.async_copy(ids1d.at[pl.ds(base, S2_SUB)], ids_s.at[dst],
                         sems[sub])
        pltpu.async_copy(p1d.at[pl.ds(base, S2_SUB)], p_s.at[dst], sems[sub])
        pltpu.async_copy(w1d.at[pl.ds(base, S2_SUB)], w_s.at[dst], sems[sub])

    pltpu.sync_copy(zpart.at[0], mv_s)
    pltpu.sync_copy(zpart.at[1], mq_s)

    def pro_z(i, _):
        ds = pl.ds(i * L, L)
        dinv_s[ds] = mv_s[ds] + mq_s[ds]
        return 0

    lax.fori_loop(0, B // L, pro_z, 0)
    pltpu.sync_copy(cpart.at[0], mv_s)
    pltpu.sync_copy(cpart.at[1], mq_s)

    def pro_c(i, _):
        ds = pl.ds(i * L, L)
        it_s[ds] = 1.0 / (mv_s[ds] + mq_s[ds] + 1.0)
        return 0

    lax.fori_loop(0, B // L, pro_c, 0)
    pltpu.sync_copy(stop, mv_s)

    def pro_d(i, _):
        ds = pl.ds(i * L, L)
        dinv_s[ds] = 1.0 / (dinv_s[ds] + jnp.exp(mv_s[ds]))
        zero = jnp.zeros((L,), jnp.float32)
        mv_s[ds] = zero
        mq_s[ds] = zero
        return 0

    lax.fori_loop(0, B // L, pro_d, 0)

    iota = lax.broadcasted_iota(jnp.int32, (L,), 0)
    nxt_idx = jnp.minimum(iota + 1, L - 1)
    zero_i = jnp.zeros((L,), jnp.int32)
    fifteen = zero_i + (L - 1)
    NV = C_PER // L

    for sub in range(N_SUB):
        dst = pl.ds(sub * S2_SUB, S2_SUB)
        for _ in range(3):
            pltpu.make_async_copy(
                ids1d.at[pl.ds(0, S2_SUB)], ids_s.at[dst], sems[sub]).wait()

    @plsc.parallel_loop(0, NV, unroll=4)
    def _(v):
        ds = pl.ds(v * L, L)
        ids16 = ids_s[ds]
        pv = p_s[ds]
        wv = w_s[ds]
        dg = plsc.load_gather(dinv_s, [ids16])
        ig = plsc.load_gather(it_s, [ids16])
        q = (1.0 - RAP) * pv * dg + RAP * ig
        cur_s = q * wv
        cur_q = q
        for d in (1, 2, 4, 8):
            idxs = jnp.maximum(iota - d, 0)
            sh_s = cur_s.at[idxs].get(mode="promise_in_bounds")
            sh_q = cur_q.at[idxs].get(mode="promise_in_bounds")
            sh_id = ids16.at[idxs].get(mode="promise_in_bounds")
            same = sh_id == ids16
            cur_q = jnp.where(same & (sh_s >= cur_s), sh_q, cur_q)
            cur_s = jnp.where(same, jnp.maximum(sh_s, cur_s), cur_s)
        nxt_id = ids16.at[nxt_idx].get(mode="promise_in_bounds")
        is_last = (ids16 != nxt_id) | (iota == L - 1)
        id0s = ids16.at[zero_i].get(mode="promise_in_bounds")
        id15s = ids16.at[fifteen].get(mode="promise_in_bounds")
        nbidx = jnp.clip(jnp.where(iota == 0, v * L - 1, v * L + L),
                         0, C_PER - 1)
        nb = plsc.load_gather(ids_s, [nbidx])
        prevs = nb.at[zero_i].get(mode="promise_in_bounds")
        nexts = nb.at[zero_i + 1].get(mode="promise_in_bounds")
        left_open = (ids16 == id0s) & (prevs == id0s)
        right_open = (ids16 == id15s) & (nexts == id15s)
        direct = is_last & (~left_open) & (~right_open)
        plsc.store_scatter(mv_s, [ids16], cur_s, mask=direct)
        plsc.store_scatter(mq_s, [ids16], cur_q, mask=direct)
        b0 = plsc.all_reduce_ffs(ids16 != id0s) - 1
        sel = jnp.where(iota == 0, b0, L - 1)
        e_s = cur_s.at[sel].get(mode="promise_in_bounds")
        e_q = cur_q.at[sel].get(mode="promise_in_bounds")
        e_i = ids16.at[sel].get(mode="promise_in_bounds")
        eidx = jnp.minimum(2 * v + iota, 2 * NV - 1)
        mask2 = iota < 2
        plsc.store_scatter(ei_s, [eidx], e_i, mask=mask2)
        plsc.store_scatter(es_s, [eidx], e_s, mask=mask2)
        plsc.store_scatter(eq_s, [eidx], e_q, mask=mask2)

    def bodyb(v, _):
        ds = pl.ds(v * L, L)
        ids16 = ei_s[ds]
        cur_s = es_s[ds]
        cur_q = eq_s[ds]
        for d in (1, 2, 4, 8):
            idxs = jnp.maximum(iota - d, 0)
            sh_s = cur_s.at[idxs].get(mode="promise_in_bounds")
            sh_q = cur_q.at[idxs].get(mode="promise_in_bounds")
            sh_id = ids16.at[idxs].get(mode="promise_in_bounds")
            same = sh_id == ids16
            cur_q = jnp.where(same & (sh_s >= cur_s), sh_q, cur_q)
            cur_s = jnp.where(same, jnp.maximum(sh_s, cur_s), cur_s)
        nxt_id = ids16.at[nxt_idx].get(mode="promise_in_bounds")
        is_last = (ids16 != nxt_id) | (iota == L - 1)
        mv = plsc.load_gather(mv_s, [ids16])
        upd = is_last & (cur_s > mv)
        plsc.store_scatter(mv_s, [ids16], cur_s, mask=upd)
        plsc.store_scatter(mq_s, [ids16], cur_q, mask=upd)
        return 0

    lax.fori_loop(0, 2 * NV // L, bodyb, 0)

    pltpu.async_copy(mv_s, bestv.at[wid], sem2)
    pltpu.async_copy(mq_s, bestq.at[wid], sem2)
    pltpu.make_async_copy(mv_s, bestv.at[wid], sem2).wait()
    pltpu.make_async_copy(mq_s, bestq.at[wid], sem2).wait()


def _e3_body(bestv_ref, bestq_ref, zpart_ref, cpart_ref, stop_ref, out_ref):
    best = bestv_ref[0]
    q = bestq_ref[0]
    for w in range(1, NW):
        v = bestv_ref[w]
        upd = v > best
        best = jnp.where(upd, v, best)
        q = jnp.where(upd, bestq_ref[w], q)
    z = zpart_ref[0] + zpart_ref[1]
    cnt = cpart_ref[0] + cpart_ref[1]
    exp_stop = jnp.exp(stop_ref[...])
    dinv = 1.0 / (z + exp_stop)
    it = 1.0 / (cnt + 1.0)
    final_stop = (1.0 - RAP) * exp_stop * dinv + RAP * it
    log_stop = jnp.log(jnp.clip(final_stop, PROB_EPS, None))
    log_edge = jnp.log(jnp.clip(q, PROB_EPS, None))
    out_ref[...] = jnp.where(final_stop >= best, log_stop, log_edge)


def _e3(bestv, bestq, zpart, cpart, stop_logits):
    rb = B // 128
    out = pl.pallas_call(
        _e3_body,
        out_shape=jax.ShapeDtypeStruct((rb, 128), jnp.float32),
    )(bestv.reshape(NW, rb, 128), bestq.reshape(NW, rb, 128),
      zpart.reshape(NC, rb, 128), cpart.reshape(NC, rb, 128),
      stop_logits.reshape(rb, 128))
    return out.reshape(B)


def kernel(edge_logits, stop_logits, edge_scores, u, edge_batch):
    p2, w2 = _e1(edge_logits, edge_scores, u)
    zpart, cpart = _s1(edge_batch, p2.reshape(E))
    bestv, bestq = _s2(edge_batch, p2.reshape(E), w2.reshape(E),
                       zpart, cpart, stop_logits)
    return _e3(bestv, bestq, zpart, cpart, stop_logits)


# speedup vs baseline: 1.3020x; 1.0091x over previous
import functools

import jax
import jax.numpy as jnp
from jax import lax
from jax.experimental import pallas as pl
from jax.experimental.pallas import tpu as pltpu
from jax.experimental.pallas import tpu_sc as plsc

RAP = 0.1
PRIOR_EPS = 1e-4
PROB_EPS = 1e-12

E = 1048576
B = 4096
NC, NS, L = 2, 16, 16
NW = NC * NS
C_PER = E // NW
ROWS_PER = C_PER // 128
S1_SUB_ROWS = 64
S2_SUB = 16384
_mesh = plsc.VectorSubcoreMesh(core_axis_name="c", subcore_axis_name="s")
_sc_params = pltpu.CompilerParams(needs_layout_passes=False)


def _e1a_body(el_ref, es_ref, p_ref):
    p_ref[...] = jnp.exp(el_ref[...]) * jnp.sqrt(
        jnp.clip(es_ref[...], PRIOR_EPS, None))


def _e1b_body(u_ref, w_ref):
    uc = jnp.clip(u_ref[...], 1e-9, 1.0 - 1e-9)
    w_ref[...] = -1.0 / jnp.log(uc)


def _e1a(edge_logits, edge_scores):
    rows = E // 128
    grid = 16
    blk = rows // grid
    spec = pl.BlockSpec((blk, 128), lambda i: (i, 0))
    return pl.pallas_call(
        _e1a_body,
        grid=(grid,),
        in_specs=[spec] * 2,
        out_specs=spec,
        out_shape=jax.ShapeDtypeStruct((rows, 128), jnp.float32),
    )(edge_logits.reshape(rows, 128), edge_scores.reshape(rows, 128))


def _e1b(u):
    rows = E // 128
    grid = 16
    blk = rows // grid
    spec = pl.BlockSpec((blk, 128), lambda i: (i, 0))
    return pl.pallas_call(
        _e1b_body,
        grid=(grid,),
        in_specs=[spec],
        out_specs=spec,
        out_shape=jax.ShapeDtypeStruct((rows, 128), jnp.float32),
    )(u.reshape(rows, 128))


def _fill(ref, n, value):
    v = jnp.full((L,), value, dtype=ref.dtype)

    def body(i, _):
        ref[pl.ds(i * L, L)] = v
        return 0

    lax.fori_loop(0, n // L, body, 0)


SL = B // NS


@functools.partial(
    pl.kernel,
    mesh=_mesh,
    out_type=[jax.ShapeDtypeStruct((NC, B), jnp.float32),
              jax.ShapeDtypeStruct((NC, B), jnp.float32)],
    scratch_types=[
        pltpu.VMEM((C_PER,), jnp.int32),
        pltpu.VMEM((C_PER,), jnp.float32),
        pltpu.VMEM((B,), jnp.float32),
        pltpu.VMEM((B,), jnp.float32),
        pltpu.VMEM((NS, SL), jnp.float32),
        pltpu.VMEM((NS, SL), jnp.float32),
        pltpu.VMEM((SL,), jnp.float32),
        pltpu.VMEM((SL,), jnp.float32),
        pltpu.VMEM_SHARED((NS, NS, SL), jnp.float32),
        pltpu.VMEM_SHARED((NS, NS, SL), jnp.float32),
        pltpu.SemaphoreType.DMA,
        pltpu.SemaphoreType.DMA,
    ],
    compiler_params=_sc_params,
)
def _s1(ids1d, p1d, zpart, cpart, ids_s, p_s, zloc, cloc, mz, mc,
        zred, cred, zsl, csl, sem0, sem1):
    c = lax.axis_index("c")
    s = lax.axis_index("s")
    wid = s * NC + c
    base = wid * C_PER

    cp0 = pltpu.async_copy(ids1d.at[pl.ds(base, C_PER)], ids_s, sem0)
    cp1 = pltpu.async_copy(p1d.at[pl.ds(base, C_PER)], p_s, sem1)
    _fill(zloc, B, 0.0)
    _fill(cloc, B, 0.0)
    cp0.wait()
    cp1.wait()

    iota = lax.broadcasted_iota(jnp.int32, (L,), 0)
    nxt_idx = jnp.minimum(iota + 1, L - 1)
    ones = jnp.ones((L,), jnp.float32)

    def one_vreg(v):
        ds = pl.ds(v * L, L)
        ids16 = ids_s[ds]
        sp = p_s[ds]
        sc = ones
        for d in (1, 2, 4, 8):
            idxs = jnp.maximum(iota - d, 0)
            sh_id = ids16.at[idxs].get(mode="promise_in_bounds")
            sh_p = sp.at[idxs].get(mode="promise_in_bounds")
            sh_c = sc.at[idxs].get(mode="promise_in_bounds")
            ok = (sh_id == ids16) & (iota >= d)
            sp = jnp.where(ok, sp + sh_p, sp)
            sc = jnp.where(ok, sc + sh_c, sc)
        nxt_id = ids16.at[nxt_idx].get(mode="promise_in_bounds")
        is_last = (ids16 != nxt_id) | (iota == L - 1)
        plsc.addupdate_scatter(zloc, [ids16], sp, mask=is_last)
        plsc.addupdate_scatter(cloc, [ids16], sc, mask=is_last)

    @plsc.parallel_loop(0, C_PER // L, unroll=4)
    def _(v):
        one_vreg(v)

    for w in range(NS):
        pltpu.sync_copy(zloc.at[pl.ds(w * SL, SL)], zsl.at[w, s])
        pltpu.sync_copy(cloc.at[pl.ds(w * SL, SL)], csl.at[w, s])
    plsc.subcore_barrier()
    pltpu.sync_copy(zsl.at[s], mz)
    pltpu.sync_copy(csl.at[s], mc)

    def red(i, _):
        ds = pl.ds(i * L, L)
        az = mz[0, ds]
        ac = mc[0, ds]
        for r in range(1, NS):
            az = az + mz[r, ds]
            ac = ac + mc[r, ds]
        zred[ds] = az
        cred[ds] = ac
        return 0

    lax.fori_loop(0, SL // L, red, 0)
    pltpu.sync_copy(zred, zpart.at[c, pl.ds(s * SL, SL)])
    pltpu.sync_copy(cred, cpart.at[c, pl.ds(s * SL, SL)])


N_SUB = C_PER // S2_SUB


@functools.partial(
    pl.kernel,
    mesh=_mesh,
    out_type=[jax.ShapeDtypeStruct((NW, B), jnp.float32),
              jax.ShapeDtypeStruct((NW, B), jnp.float32)],
    scratch_types=[
        pltpu.VMEM((B,), jnp.float32),
        pltpu.VMEM((B,), jnp.float32),
        pltpu.VMEM((B,), jnp.float32),
        pltpu.VMEM((B,), jnp.float32),
        pltpu.VMEM((C_PER,), jnp.int32),
        pltpu.VMEM((C_PER,), jnp.float32),
        pltpu.VMEM((C_PER,), jnp.float32),
        pltpu.VMEM((2 * C_PER // L,), jnp.int32),
        pltpu.VMEM((2 * C_PER // L,), jnp.float32),
        pltpu.VMEM((2 * C_PER // L,), jnp.float32),
        pltpu.SemaphoreType.DMA,
        pltpu.SemaphoreType.DMA,
        pltpu.SemaphoreType.DMA,
    ],
    compiler_params=_sc_params,
)
def _s2(ids1d, p1d, w1d, zpart, cpart, stop, bestv, bestq,
        dinv_s, it_s, mv_s, mq_s, ids_s, p_s, w_s, ei_s, es_s, eq_s,
        sem0, sem1, sem2):
    c = lax.axis_index("c")
    s = lax.axis_index("s")
    wid = s * NC + c

    sems = (sem0, sem1)
    for sub in range(N_SUB):
        base = wid * C_PER + sub * S2_SUB
        dst = pl.ds(sub * S2_SUB, S2_SUB)
        pltpu.async_copy(ids1d.at[pl.ds(base, S2_SUB)], ids_s.at[dst],
                         sems[sub])
        pltpu.async_copy(p1d.at[pl.ds(base, S2_SUB)], p_s.at[dst], sems[sub])
        pltpu.async_copy(w1d.at[pl.ds(base, S2_SUB)], w_s.at[dst], sems[sub])

    pltpu.sync_copy(zpart.at[0], mv_s)
    pltpu.sync_copy(zpart.at[1], mq_s)

    def pro_z(i, _):
        ds = pl.ds(i * L, L)
        dinv_s[ds] = mv_s[ds] + mq_s[ds]
        return 0

    lax.fori_loop(0, B // L, pro_z, 0)
    pltpu.sync_copy(cpart.at[0], mv_s)
    pltpu.sync_copy(cpart.at[1], mq_s)

    def pro_c(i, _):
        ds = pl.ds(i * L, L)
        it_s[ds] = 1.0 / (mv_s[ds] + mq_s[ds] + 1.0)
        return 0

    lax.fori_loop(0, B // L, pro_c, 0)
    pltpu.sync_copy(stop, mv_s)

    def pro_d(i, _):
        ds = pl.ds(i * L, L)
        dinv_s[ds] = 1.0 / (dinv_s[ds] + jnp.exp(mv_s[ds]))
        zero = jnp.zeros((L,), jnp.float32)
        mv_s[ds] = zero
        mq_s[ds] = zero
        return 0

    lax.fori_loop(0, B // L, pro_d, 0)

    iota = lax.broadcasted_iota(jnp.int32, (L,), 0)
    nxt_idx = jnp.minimum(iota + 1, L - 1)
    zero_i = jnp.zeros((L,), jnp.int32)
    fifteen = zero_i + (L - 1)
    NV = C_PER // L

    for sub in range(N_SUB):
        dst = pl.ds(sub * S2_SUB, S2_SUB)
        for _ in range(3):
            pltpu.make_async_copy(
                ids1d.at[pl.ds(0, S2_SUB)], ids_s.at[dst], sems[sub]).wait()

    @plsc.parallel_loop(0, NV, unroll=4)
    def _(v):
        ds = pl.ds(v * L, L)
        ids16 = ids_s[ds]
        pv = p_s[ds]
        wv = w_s[ds]
        dg = plsc.load_gather(dinv_s, [ids16])
        ig = plsc.load_gather(it_s, [ids16])
        q = (1.0 - RAP) * pv * dg + RAP * ig
        cur_s = q * wv
        cur_q = q
        for d in (1, 2, 4, 8):
            idxs = jnp.maximum(iota - d, 0)
            sh_s = cur_s.at[idxs].get(mode="promise_in_bounds")
            sh_q = cur_q.at[idxs].get(mode="promise_in_bounds")
            sh_id = ids16.at[idxs].get(mode="promise_in_bounds")
            same = sh_id == ids16
            cur_q = jnp.where(same & (sh_s >= cur_s), sh_q, cur_q)
            cur_s = jnp.where(same, jnp.maximum(sh_s, cur_s), cur_s)
        nxt_id = ids16.at[nxt_idx].get(mode="promise_in_bounds")
        is_last = (ids16 != nxt_id) | (iota == L - 1)
        id0s = ids16.at[zero_i].get(mode="promise_in_bounds")
        id15s = ids16.at[fifteen].get(mode="promise_in_bounds")
        nbidx = jnp.clip(jnp.where(iota == 0, v * L - 1, v * L + L),
                         0, C_PER - 1)
        nb = plsc.load_gather(ids_s, [nbidx])
        prevs = nb.at[zero_i].get(mode="promise_in_bounds")
        nexts = nb.at[zero_i + 1].get(mode="promise_in_bounds")
        left_open = (ids16 == id0s) & (prevs == id0s)
        right_open = (ids16 == id15s) & (nexts == id15s)
        direct = is_last & (~left_open) & (~right_open)
        plsc.store_scatter(mv_s, [ids16], cur_s, mask=direct)
        plsc.store_scatter(mq_s, [ids16], cur_q, mask=direct)
        b0 = plsc.all_reduce_ffs(ids16 != id0s) - 1
        sel = jnp.where(iota == 0, b0, L - 1)
        e_s = cur_s.at[sel].get(mode="promise_in_bounds")
        e_q = cur_q.at[sel].get(mode="promise_in_bounds")
        e_i = ids16.at[sel].get(mode="promise_in_bounds")
        eidx = jnp.minimum(2 * v + iota, 2 * NV - 1)
        mask2 = iota < 2
        plsc.store_scatter(ei_s, [eidx], e_i, mask=mask2)
        plsc.store_scatter(es_s, [eidx], e_s, mask=mask2)
        plsc.store_scatter(eq_s, [eidx], e_q, mask=mask2)

    def bodyb(v, _):
        ds = pl.ds(v * L, L)
        ids16 = ei_s[ds]
        cur_s = es_s[ds]
        cur_q = eq_s[ds]
        for d in (1, 2, 4, 8):
            idxs = jnp.maximum(iota - d, 0)
            sh_s = cur_s.at[idxs].get(mode="promise_in_bounds")
            sh_q = cur_q.at[idxs].get(mode="promise_in_bounds")
            sh_id = ids16.at[idxs].get(mode="promise_in_bounds")
            same = sh_id == ids16
            cur_q = jnp.where(same & (sh_s >= cur_s), sh_q, cur_q)
            cur_s = jnp.where(same, jnp.maximum(sh_s, cur_s), cur_s)
        nxt_id = ids16.at[nxt_idx].get(mode="promise_in_bounds")
        is_last = (ids16 != nxt_id) | (iota == L - 1)
        mv = plsc.load_gather(mv_s, [ids16])
        upd = is_last & (cur_s > mv)
        plsc.store_scatter(mv_s, [ids16], cur_s, mask=upd)
        plsc.store_scatter(mq_s, [ids16], cur_q, mask=upd)
        return 0

    lax.fori_loop(0, 2 * NV // L, bodyb, 0)

    pltpu.async_copy(mv_s, bestv.at[wid], sem2)
    pltpu.async_copy(mq_s, bestq.at[wid], sem2)
    pltpu.make_async_copy(mv_s, bestv.at[wid], sem2).wait()
    pltpu.make_async_copy(mq_s, bestq.at[wid], sem2).wait()


def _e3_body(bestv_ref, bestq_ref, zpart_ref, cpart_ref, stop_ref, out_ref):
    best = bestv_ref[0]
    q = bestq_ref[0]
    for w in range(1, NW):
        v = bestv_ref[w]
        upd = v > best
        best = jnp.where(upd, v, best)
        q = jnp.where(upd, bestq_ref[w], q)
    z = zpart_ref[0] + zpart_ref[1]
    cnt = cpart_ref[0] + cpart_ref[1]
    exp_stop = jnp.exp(stop_ref[...])
    dinv = 1.0 / (z + exp_stop)
    it = 1.0 / (cnt + 1.0)
    final_stop = (1.0 - RAP) * exp_stop * dinv + RAP * it
    log_stop = jnp.log(jnp.clip(final_stop, PROB_EPS, None))
    log_edge = jnp.log(jnp.clip(q, PROB_EPS, None))
    out_ref[...] = jnp.where(final_stop >= best, log_stop, log_edge)


def _e3(bestv, bestq, zpart, cpart, stop_logits):
    rb = B // 128
    out = pl.pallas_call(
        _e3_body,
        out_shape=jax.ShapeDtypeStruct((rb, 128), jnp.float32),
    )(bestv.reshape(NW, rb, 128), bestq.reshape(NW, rb, 128),
      zpart.reshape(NC, rb, 128), cpart.reshape(NC, rb, 128),
      stop_logits.reshape(rb, 128))
    return out.reshape(B)


def kernel(edge_logits, stop_logits, edge_scores, u, edge_batch):
    p2 = _e1a(edge_logits, edge_scores)
    zpart, cpart = _s1(edge_batch, p2.reshape(E))
    w2 = _e1b(u)
    bestv, bestq = _s2(edge_batch, p2.reshape(E), w2.reshape(E),
                       zpart, cpart, stop_logits)
    return _e3(bestv, bestq, zpart, cpart, stop_logits)


# speedup vs baseline: 1.3487x; 1.0358x over previous
import functools

import jax
import jax.numpy as jnp
from jax import lax
from jax.experimental import pallas as pl
from jax.experimental.pallas import tpu as pltpu
from jax.experimental.pallas import tpu_sc as plsc

RAP = 0.1
PRIOR_EPS = 1e-4
PROB_EPS = 1e-12

E = 1048576
B = 4096
NC, NS, L = 2, 16, 16
NW = NC * NS
C_PER = E // NW
ROWS_PER = C_PER // 128
S1_SUB_ROWS = 64
S2_SUB = 16384
_mesh = plsc.VectorSubcoreMesh(core_axis_name="c", subcore_axis_name="s")
_sc_params = pltpu.CompilerParams(needs_layout_passes=False)


def _e1a_body(el_ref, es_ref, p_ref):
    p_ref[...] = jnp.exp(el_ref[...]) * jnp.sqrt(
        jnp.clip(es_ref[...], PRIOR_EPS, None))


def _e1b_body(u_ref, w_ref):
    uc = jnp.clip(u_ref[...], 1e-9, 1.0 - 1e-9)
    w_ref[...] = -1.0 / jnp.log(uc)


def _e1a(edge_logits, edge_scores):
    rows = E // 128
    grid = 16
    blk = rows // grid
    spec = pl.BlockSpec((blk, 128), lambda i: (i, 0))
    return pl.pallas_call(
        _e1a_body,
        grid=(grid,),
        in_specs=[spec] * 2,
        out_specs=spec,
        out_shape=jax.ShapeDtypeStruct((rows, 128), jnp.float32),
    )(edge_logits.reshape(rows, 128), edge_scores.reshape(rows, 128))


def _e1b(u):
    rows = E // 128
    grid = 16
    blk = rows // grid
    spec = pl.BlockSpec((blk, 128), lambda i: (i, 0))
    return pl.pallas_call(
        _e1b_body,
        grid=(grid,),
        in_specs=[spec],
        out_specs=spec,
        out_shape=jax.ShapeDtypeStruct((rows, 128), jnp.float32),
    )(u.reshape(rows, 128))


def _fill(ref, n, value):
    v = jnp.full((L,), value, dtype=ref.dtype)

    def body(i, _):
        ref[pl.ds(i * L, L)] = v
        return 0

    lax.fori_loop(0, n // L, body, 0)


SL = B // NS


@functools.partial(
    pl.kernel,
    mesh=_mesh,
    out_type=[jax.ShapeDtypeStruct((NC, B), jnp.float32),
              jax.ShapeDtypeStruct((NC, B), jnp.float32)],
    scratch_types=[
        pltpu.VMEM((C_PER,), jnp.int32),
        pltpu.VMEM((C_PER,), jnp.float32),
        pltpu.VMEM((B,), jnp.float32),
        pltpu.VMEM((B,), jnp.float32),
        pltpu.VMEM((NS, SL), jnp.float32),
        pltpu.VMEM((NS, SL), jnp.float32),
        pltpu.VMEM((SL,), jnp.float32),
        pltpu.VMEM((SL,), jnp.float32),
        pltpu.VMEM_SHARED((NS, NS, SL), jnp.float32),
        pltpu.VMEM_SHARED((NS, NS, SL), jnp.float32),
        pltpu.SemaphoreType.DMA,
        pltpu.SemaphoreType.DMA,
    ],
    compiler_params=_sc_params,
)
def _s1(ids1d, p1d, zpart, cpart, ids_s, p_s, zloc, cloc, mz, mc,
        zred, cred, zsl, csl, sem0, sem1):
    c = lax.axis_index("c")
    s = lax.axis_index("s")
    wid = s * NC + c
    base = wid * C_PER

    cp0 = pltpu.async_copy(ids1d.at[pl.ds(base, C_PER)], ids_s, sem0)
    cp1 = pltpu.async_copy(p1d.at[pl.ds(base, C_PER)], p_s, sem1)
    _fill(zloc, B, 0.0)
    _fill(cloc, B, 0.0)
    cp0.wait()
    cp1.wait()

    iota = lax.broadcasted_iota(jnp.int32, (L,), 0)
    nxt_idx = jnp.minimum(iota + 1, L - 1)
    ones = jnp.ones((L,), jnp.float32)

    def one_vreg(v):
        ds = pl.ds(v * L, L)
        ids16 = ids_s[ds]
        sp = p_s[ds]
        sc = ones
        for d in (1, 2, 4, 8):
            idxs = jnp.maximum(iota - d, 0)
            sh_id = ids16.at[idxs].get(mode="promise_in_bounds")
            sh_p = sp.at[idxs].get(mode="promise_in_bounds")
            sh_c = sc.at[idxs].get(mode="promise_in_bounds")
            ok = (sh_id == ids16) & (iota >= d)
            sp = jnp.where(ok, sp + sh_p, sp)
            sc = jnp.where(ok, sc + sh_c, sc)
        nxt_id = ids16.at[nxt_idx].get(mode="promise_in_bounds")
        is_last = (ids16 != nxt_id) | (iota == L - 1)
        plsc.addupdate_scatter(zloc, [ids16], sp, mask=is_last)
        plsc.addupdate_scatter(cloc, [ids16], sc, mask=is_last)

    @plsc.parallel_loop(0, C_PER // L, unroll=4)
    def _(v):
        one_vreg(v)

    for w in range(NS):
        pltpu.sync_copy(zloc.at[pl.ds(w * SL, SL)], zsl.at[w, s])
        pltpu.sync_copy(cloc.at[pl.ds(w * SL, SL)], csl.at[w, s])
    plsc.subcore_barrier()
    pltpu.sync_copy(zsl.at[s], mz)
    pltpu.sync_copy(csl.at[s], mc)

    def red(i, _):
        ds = pl.ds(i * L, L)
        az = mz[0, ds]
        ac = mc[0, ds]
        for r in range(1, NS):
            az = az + mz[r, ds]
            ac = ac + mc[r, ds]
        zred[ds] = az
        cred[ds] = ac
        return 0

    lax.fori_loop(0, SL // L, red, 0)
    pltpu.sync_copy(zred, zpart.at[c, pl.ds(s * SL, SL)])
    pltpu.sync_copy(cred, cpart.at[c, pl.ds(s * SL, SL)])


N_SUB = C_PER // S2_SUB


@functools.partial(
    pl.kernel,
    mesh=_mesh,
    out_type=[jax.ShapeDtypeStruct((NW, B), jnp.float32),
              jax.ShapeDtypeStruct((NW, B), jnp.float32)],
    scratch_types=[
        pltpu.VMEM((B,), jnp.float32),
        pltpu.VMEM((B,), jnp.float32),
        pltpu.VMEM((B,), jnp.float32),
        pltpu.VMEM((B,), jnp.float32),
        pltpu.VMEM((C_PER,), jnp.int32),
        pltpu.VMEM((C_PER,), jnp.float32),
        pltpu.VMEM((C_PER,), jnp.float32),
        pltpu.VMEM((2 * C_PER // L,), jnp.int32),
        pltpu.VMEM((2 * C_PER // L,), jnp.float32),
        pltpu.VMEM((2 * C_PER // L,), jnp.float32),
        pltpu.SemaphoreType.DMA,
        pltpu.SemaphoreType.DMA,
        pltpu.SemaphoreType.DMA,
    ],
    compiler_params=_sc_params,
)
def _s2(ids1d, p1d, w1d, zpart, cpart, stop, bestv, bestq,
        dinv_s, it_s, mv_s, mq_s, ids_s, p_s, w_s, ei_s, es_s, eq_s,
        sem0, sem1, sem2):
    c = lax.axis_index("c")
    s = lax.axis_index("s")
    wid = s * NC + c

    sems = (sem0, sem1)
    for sub in range(N_SUB):
        base = wid * C_PER + sub * S2_SUB
        dst = pl.ds(sub * S2_SUB, S2_SUB)
        pltpu.async_copy(ids1d.at[pl.ds(base, S2_SUB)], ids_s.at[dst],
                         sems[sub])
        pltpu.async_copy(p1d.at[pl.ds(base, S2_SUB)], p_s.at[dst], sems[sub])
        pltpu.async_copy(w1d.at[pl.ds(base, S2_SUB)], w_s.at[dst], sems[sub])

    pltpu.sync_copy(zpart.at[0], mv_s)
    pltpu.sync_copy(zpart.at[1], mq_s)

    def pro_z(i, _):
        ds = pl.ds(i * L, L)
        dinv_s[ds] = mv_s[ds] + mq_s[ds]
        return 0

    lax.fori_loop(0, B // L, pro_z, 0)
    pltpu.sync_copy(cpart.at[0], mv_s)
    pltpu.sync_copy(cpart.at[1], mq_s)

    def pro_c(i, _):
        ds = pl.ds(i * L, L)
        it_s[ds] = 1.0 / (mv_s[ds] + mq_s[ds] + 1.0)
        return 0

    lax.fori_loop(0, B // L, pro_c, 0)
    pltpu.sync_copy(stop, mv_s)

    def pro_d(i, _):
        ds = pl.ds(i * L, L)
        dinv_s[ds] = 1.0 / (dinv_s[ds] + jnp.exp(mv_s[ds]))
        zero = jnp.zeros((L,), jnp.float32)
        mv_s[ds] = zero
        mq_s[ds] = zero
        return 0

    lax.fori_loop(0, B // L, pro_d, 0)

    iota = lax.broadcasted_iota(jnp.int32, (L,), 0)
    nxt_idx = jnp.minimum(iota + 1, L - 1)
    zero_i = jnp.zeros((L,), jnp.int32)
    fifteen = zero_i + (L - 1)
    NV = C_PER // L

    for sub in range(N_SUB):
        dst = pl.ds(sub * S2_SUB, S2_SUB)
        for _ in range(3):
            pltpu.make_async_copy(
                ids1d.at[pl.ds(0, S2_SUB)], ids_s.at[dst], sems[sub]).wait()

    @plsc.parallel_loop(0, NV, unroll=4)
    def _(v):
        ds = pl.ds(v * L, L)
        ids16 = ids_s[ds]
        pv = p_s[ds]
        wv = w_s[ds]
        dg = plsc.load_gather(dinv_s, [ids16])
        ig = plsc.load_gather(it_s, [ids16])
        q = (1.0 - RAP) * pv * dg + RAP * ig
        sbits = plsc.bitcast(q * wv, jnp.int32)
        cur_s = plsc.bitcast((sbits & ~0xF) | (L - 1 - iota), jnp.float32)
        for d in (1, 2, 4, 8):
            idxs = jnp.maximum(iota - d, 0)
            sh_s = cur_s.at[idxs].get(mode="promise_in_bounds")
            sh_id = ids16.at[idxs].get(mode="promise_in_bounds")
            same = sh_id == ids16
            cur_s = jnp.where(same, jnp.maximum(sh_s, cur_s), cur_s)
        win = (L - 1) - (plsc.bitcast(cur_s, jnp.int32) & 0xF)
        q_win = q.at[win].get(mode="promise_in_bounds")
        nxt_id = ids16.at[nxt_idx].get(mode="promise_in_bounds")
        is_last = (ids16 != nxt_id) | (iota == L - 1)
        id0s = ids16.at[zero_i].get(mode="promise_in_bounds")
        id15s = ids16.at[fifteen].get(mode="promise_in_bounds")
        nbidx = jnp.clip(jnp.where(iota == 0, v * L - 1, v * L + L),
                         0, C_PER - 1)
        nb = plsc.load_gather(ids_s, [nbidx])
        prevs = nb.at[zero_i].get(mode="promise_in_bounds")
        nexts = nb.at[zero_i + 1].get(mode="promise_in_bounds")
        left_open = (ids16 == id0s) & (prevs == id0s)
        right_open = (ids16 == id15s) & (nexts == id15s)
        direct = is_last & (~left_open) & (~right_open)
        plsc.store_scatter(mv_s, [ids16], cur_s, mask=direct)
        plsc.store_scatter(mq_s, [ids16], q_win, mask=direct)
        b0 = plsc.all_reduce_ffs(ids16 != id0s) - 1
        sel = jnp.where(iota == 0, b0, L - 1)
        e_s = cur_s.at[sel].get(mode="promise_in_bounds")
        e_q = q_win.at[sel].get(mode="promise_in_bounds")
        e_i = ids16.at[sel].get(mode="promise_in_bounds")
        eidx = jnp.minimum(2 * v + iota, 2 * NV - 1)
        mask2 = iota < 2
        plsc.store_scatter(ei_s, [eidx], e_i, mask=mask2)
        plsc.store_scatter(es_s, [eidx], e_s, mask=mask2)
        plsc.store_scatter(eq_s, [eidx], e_q, mask=mask2)

    def bodyb(v, _):
        ds = pl.ds(v * L, L)
        ids16 = ei_s[ds]
        cur_s = es_s[ds]
        cur_q = eq_s[ds]
        for d in (1, 2, 4, 8):
            idxs = jnp.maximum(iota - d, 0)
            sh_s = cur_s.at[idxs].get(mode="promise_in_bounds")
            sh_q = cur_q.at[idxs].get(mode="promise_in_bounds")
            sh_id = ids16.at[idxs].get(mode="promise_in_bounds")
            same = sh_id == ids16
            cur_q = jnp.where(same & (sh_s >= cur_s), sh_q, cur_q)
            cur_s = jnp.where(same, jnp.maximum(sh_s, cur_s), cur_s)
        nxt_id = ids16.at[nxt_idx].get(mode="promise_in_bounds")
        is_last = (ids16 != nxt_id) | (iota == L - 1)
        mv = plsc.load_gather(mv_s, [ids16])
        upd = is_last & (cur_s > mv)
        plsc.store_scatter(mv_s, [ids16], cur_s, mask=upd)
        plsc.store_scatter(mq_s, [ids16], cur_q, mask=upd)
        return 0

    lax.fori_loop(0, 2 * NV // L, bodyb, 0)

    pltpu.async_copy(mv_s, bestv.at[wid], sem2)
    pltpu.async_copy(mq_s, bestq.at[wid], sem2)
    pltpu.make_async_copy(mv_s, bestv.at[wid], sem2).wait()
    pltpu.make_async_copy(mq_s, bestq.at[wid], sem2).wait()


def _e3_body(bestv_ref, bestq_ref, zpart_ref, cpart_ref, stop_ref, out_ref):
    best = bestv_ref[0]
    q = bestq_ref[0]
    for w in range(1, NW):
        v = bestv_ref[w]
        upd = v > best
        best = jnp.where(upd, v, best)
        q = jnp.where(upd, bestq_ref[w], q)
    z = zpart_ref[0] + zpart_ref[1]
    cnt = cpart_ref[0] + cpart_ref[1]
    exp_stop = jnp.exp(stop_ref[...])
    dinv = 1.0 / (z + exp_stop)
    it = 1.0 / (cnt + 1.0)
    final_stop = (1.0 - RAP) * exp_stop * dinv + RAP * it
    log_stop = jnp.log(jnp.clip(final_stop, PROB_EPS, None))
    log_edge = jnp.log(jnp.clip(q, PROB_EPS, None))
    out_ref[...] = jnp.where(final_stop >= best, log_stop, log_edge)


def _e3(bestv, bestq, zpart, cpart, stop_logits):
    rb = B // 128
    out = pl.pallas_call(
        _e3_body,
        out_shape=jax.ShapeDtypeStruct((rb, 128), jnp.float32),
    )(bestv.reshape(NW, rb, 128), bestq.reshape(NW, rb, 128),
      zpart.reshape(NC, rb, 128), cpart.reshape(NC, rb, 128),
      stop_logits.reshape(rb, 128))
    return out.reshape(B)


def kernel(edge_logits, stop_logits, edge_scores, u, edge_batch):
    p2 = _e1a(edge_logits, edge_scores)
    zpart, cpart = _s1(edge_batch, p2.reshape(E))
    w2 = _e1b(u)
    bestv, bestq = _s2(edge_batch, p2.reshape(E), w2.reshape(E),
                       zpart, cpart, stop_logits)
    return _e3(bestv, bestq, zpart, cpart, stop_logits)


# speedup vs baseline: 1.4604x; 1.0828x over previous
import functools

import jax
import jax.numpy as jnp
from jax import lax
from jax.experimental import pallas as pl
from jax.experimental.pallas import tpu as pltpu
from jax.experimental.pallas import tpu_sc as plsc

RAP = 0.1
PRIOR_EPS = 1e-4
PROB_EPS = 1e-12

E = 1048576
B = 4096
NC, NS, L = 2, 16, 16
NW = NC * NS
C_PER = E // NW
ROWS_PER = C_PER // 128
S1_SUB_ROWS = 64
S2_SUB = 16384
_mesh = plsc.VectorSubcoreMesh(core_axis_name="c", subcore_axis_name="s")
_sc_params = pltpu.CompilerParams(needs_layout_passes=False)


def _e1a_body(el_ref, es_ref, p_ref):
    p_ref[...] = jnp.exp(el_ref[...]) * jnp.sqrt(
        jnp.clip(es_ref[...], PRIOR_EPS, None))


def _e1b_body(u_ref, w_ref):
    uc = jnp.clip(u_ref[...], 1e-9, 1.0 - 1e-9)
    w_ref[...] = -1.0 / jnp.log(uc)


def _e1a(edge_logits, edge_scores):
    rows = E // 128
    grid = 16
    blk = rows // grid
    spec = pl.BlockSpec((blk, 128), lambda i: (i, 0))
    return pl.pallas_call(
        _e1a_body,
        grid=(grid,),
        in_specs=[spec] * 2,
        out_specs=spec,
        out_shape=jax.ShapeDtypeStruct((rows, 128), jnp.float32),
    )(edge_logits.reshape(rows, 128), edge_scores.reshape(rows, 128))


def _e1b(u):
    rows = E // 128
    grid = 16
    blk = rows // grid
    spec = pl.BlockSpec((blk, 128), lambda i: (i, 0))
    return pl.pallas_call(
        _e1b_body,
        grid=(grid,),
        in_specs=[spec],
        out_specs=spec,
        out_shape=jax.ShapeDtypeStruct((rows, 128), jnp.float32),
    )(u.reshape(rows, 128))


def _fill(ref, n, value):
    v = jnp.full((L,), value, dtype=ref.dtype)

    def body(i, _):
        ref[pl.ds(i * L, L)] = v
        return 0

    lax.fori_loop(0, n // L, body, 0)


SL = B // NS


@functools.partial(
    pl.kernel,
    mesh=_mesh,
    out_type=[jax.ShapeDtypeStruct((NC, B), jnp.float32),
              jax.ShapeDtypeStruct((NC, B), jnp.float32)],
    scratch_types=[
        pltpu.VMEM((C_PER,), jnp.int32),
        pltpu.VMEM((C_PER,), jnp.float32),
        pltpu.VMEM((B,), jnp.float32),
        pltpu.VMEM((B,), jnp.float32),
        pltpu.VMEM((NS, SL), jnp.float32),
        pltpu.VMEM((NS, SL), jnp.float32),
        pltpu.VMEM((SL,), jnp.float32),
        pltpu.VMEM((SL,), jnp.float32),
        pltpu.VMEM_SHARED((NS, NS, SL), jnp.float32),
        pltpu.VMEM_SHARED((NS, NS, SL), jnp.float32),
        pltpu.SemaphoreType.DMA,
        pltpu.SemaphoreType.DMA,
    ],
    compiler_params=_sc_params,
)
def _s1(ids1d, p1d, zpart, cpart, ids_s, p_s, zloc, cloc, mz, mc,
        zred, cred, zsl, csl, sem0, sem1):
    c = lax.axis_index("c")
    s = lax.axis_index("s")
    wid = s * NC + c
    base = wid * C_PER

    cp0 = pltpu.async_copy(ids1d.at[pl.ds(base, C_PER)], ids_s, sem0)
    cp1 = pltpu.async_copy(p1d.at[pl.ds(base, C_PER)], p_s, sem1)
    _fill(zloc, B, 0.0)
    _fill(cloc, B, 0.0)
    cp0.wait()
    cp1.wait()

    iota = lax.broadcasted_iota(jnp.int32, (L,), 0)
    nxt_idx = jnp.minimum(iota + 1, L - 1)
    ones = jnp.ones((L,), jnp.float32)

    def one_vreg(v):
        ds = pl.ds(v * L, L)
        ids16 = ids_s[ds]
        sp = p_s[ds]
        sc = ones
        for d in (1, 2, 4, 8):
            idxs = jnp.maximum(iota - d, 0)
            sh_id = ids16.at[idxs].get(mode="promise_in_bounds")
            sh_p = sp.at[idxs].get(mode="promise_in_bounds")
            sh_c = sc.at[idxs].get(mode="promise_in_bounds")
            ok = (sh_id == ids16) & (iota >= d)
            sp = jnp.where(ok, sp + sh_p, sp)
            sc = jnp.where(ok, sc + sh_c, sc)
        nxt_id = ids16.at[nxt_idx].get(mode="promise_in_bounds")
        is_last = (ids16 != nxt_id) | (iota == L - 1)
        plsc.addupdate_scatter(zloc, [ids16], sp, mask=is_last)
        plsc.addupdate_scatter(cloc, [ids16], sc, mask=is_last)

    @plsc.parallel_loop(0, C_PER // L, unroll=4)
    def _(v):
        one_vreg(v)

    for w in range(NS):
        pltpu.sync_copy(zloc.at[pl.ds(w * SL, SL)], zsl.at[w, s])
        pltpu.sync_copy(cloc.at[pl.ds(w * SL, SL)], csl.at[w, s])
    plsc.subcore_barrier()
    pltpu.sync_copy(zsl.at[s], mz)
    pltpu.sync_copy(csl.at[s], mc)

    def red(i, _):
        ds = pl.ds(i * L, L)
        az = mz[0, ds]
        ac = mc[0, ds]
        for r in range(1, NS):
            az = az + mz[r, ds]
            ac = ac + mc[r, ds]
        zred[ds] = az
        cred[ds] = ac
        return 0

    lax.fori_loop(0, SL // L, red, 0)
    pltpu.sync_copy(zred, zpart.at[c, pl.ds(s * SL, SL)])
    pltpu.sync_copy(cred, cpart.at[c, pl.ds(s * SL, SL)])


N_SUB = C_PER // S2_SUB


@functools.partial(
    pl.kernel,
    mesh=_mesh,
    out_type=[jax.ShapeDtypeStruct((NW, B), jnp.float32),
              jax.ShapeDtypeStruct((NW, B), jnp.float32)],
    scratch_types=[
        pltpu.VMEM((B,), jnp.float32),
        pltpu.VMEM((B,), jnp.float32),
        pltpu.VMEM((B,), jnp.float32),
        pltpu.VMEM((B,), jnp.float32),
        pltpu.VMEM((C_PER + 2 * L,), jnp.int32),
        pltpu.VMEM((C_PER,), jnp.float32),
        pltpu.VMEM((C_PER,), jnp.float32),
        pltpu.VMEM((2 * C_PER // L,), jnp.int32),
        pltpu.VMEM((2 * C_PER // L,), jnp.float32),
        pltpu.VMEM((2 * C_PER // L,), jnp.float32),
        pltpu.SemaphoreType.DMA,
        pltpu.SemaphoreType.DMA,
        pltpu.SemaphoreType.DMA,
    ],
    compiler_params=_sc_params,
)
def _s2(ids1d, p1d, w1d, zpart, cpart, stop, bestv, bestq,
        dinv_s, it_s, mv_s, mq_s, ids_s, p_s, w_s, ei_s, es_s, eq_s,
        sem0, sem1, sem2):
    c = lax.axis_index("c")
    s = lax.axis_index("s")
    wid = s * NC + c

    sems = (sem0, sem1)
    for sub in range(N_SUB):
        base = wid * C_PER + sub * S2_SUB
        pltpu.async_copy(ids1d.at[pl.ds(base, S2_SUB)],
                         ids_s.at[pl.ds(L + sub * S2_SUB, S2_SUB)], sems[sub])
        dst = pl.ds(sub * S2_SUB, S2_SUB)
        pltpu.async_copy(p1d.at[pl.ds(base, S2_SUB)], p_s.at[dst], sems[sub])
        pltpu.async_copy(w1d.at[pl.ds(base, S2_SUB)], w_s.at[dst], sems[sub])
    ids_s[pl.ds(0, L)] = jnp.full((L,), -1, jnp.int32)
    ids_s[pl.ds(L + C_PER, L)] = jnp.full((L,), -1, jnp.int32)

    pltpu.sync_copy(zpart.at[0], mv_s)
    pltpu.sync_copy(zpart.at[1], mq_s)

    def pro_z(i, _):
        ds = pl.ds(i * L, L)
        dinv_s[ds] = mv_s[ds] + mq_s[ds]
        return 0

    lax.fori_loop(0, B // L, pro_z, 0)
    pltpu.sync_copy(cpart.at[0], mv_s)
    pltpu.sync_copy(cpart.at[1], mq_s)

    def pro_c(i, _):
        ds = pl.ds(i * L, L)
        it_s[ds] = 1.0 / (mv_s[ds] + mq_s[ds] + 1.0)
        return 0

    lax.fori_loop(0, B // L, pro_c, 0)
    pltpu.sync_copy(stop, mv_s)

    def pro_d(i, _):
        ds = pl.ds(i * L, L)
        dinv_s[ds] = 1.0 / (dinv_s[ds] + jnp.exp(mv_s[ds]))
        zero = jnp.zeros((L,), jnp.float32)
        mv_s[ds] = zero
        mq_s[ds] = zero
        return 0

    lax.fori_loop(0, B // L, pro_d, 0)

    iota = lax.broadcasted_iota(jnp.int32, (L,), 0)
    nxt_idx = jnp.minimum(iota + 1, L - 1)
    zero_i = jnp.zeros((L,), jnp.int32)
    fifteen = zero_i + (L - 1)
    NV = C_PER // L

    for sub in range(N_SUB):
        dst = pl.ds(sub * S2_SUB, S2_SUB)
        for _ in range(3):
            pltpu.make_async_copy(
                ids1d.at[pl.ds(0, S2_SUB)], p_s.at[dst], sems[sub]).wait()

    @plsc.parallel_loop(0, NV, unroll=4)
    def _(v):
        b16 = L + v * L
        ids16 = ids_s[pl.ds(b16, L)]
        pv = p_s[pl.ds(v * L, L)]
        wv = w_s[pl.ds(v * L, L)]
        dg = plsc.load_gather(dinv_s, [ids16])
        ig = plsc.load_gather(it_s, [ids16])
        q = (1.0 - RAP) * pv * dg + RAP * ig
        sbits = plsc.bitcast(q * wv, jnp.int32)
        cur_s = plsc.bitcast((sbits & ~0xF) | (L - 1 - iota), jnp.float32)
        prv1 = ids_s[pl.ds(b16 - 1, L)]
        for d in (1, 2, 4, 8):
            idxs = jnp.maximum(iota - d, 0)
            sh_s = cur_s.at[idxs].get(mode="promise_in_bounds")
            sh_id = prv1 if d == 1 else ids_s[pl.ds(b16 - d, L)]
            same = sh_id == ids16
            cur_s = jnp.where(same, jnp.maximum(sh_s, cur_s), cur_s)
        win = (L - 1) - (plsc.bitcast(cur_s, jnp.int32) & 0xF)
        q_win = q.at[win].get(mode="promise_in_bounds")
        is_last = ids16 != ids_s[pl.ds(b16 + 1, L)]
        id0 = ids16[0]
        id0s = zero_i + id0
        left_open = (ids16 == id0s) & (prv1[0] == id0)
        direct = is_last & (~left_open)
        plsc.store_scatter(mv_s, [ids16], cur_s, mask=direct)
        plsc.store_scatter(mq_s, [ids16], q_win, mask=direct)
        b0 = plsc.all_reduce_ffs(ids16 != id0s) - 1
        sel = jnp.where(iota == 0, b0, L - 1)
        e_s = cur_s.at[sel].get(mode="promise_in_bounds")
        e_q = q_win.at[sel].get(mode="promise_in_bounds")
        e_i = ids16.at[sel].get(mode="promise_in_bounds")
        eidx = jnp.minimum(2 * v + iota, 2 * NV - 1)
        mask2 = iota < 2
        plsc.store_scatter(ei_s, [eidx], e_i, mask=mask2)
        plsc.store_scatter(es_s, [eidx], e_s, mask=mask2)
        plsc.store_scatter(eq_s, [eidx], e_q, mask=mask2)

    def bodyb(v, _):
        ds = pl.ds(v * L, L)
        ids16 = ei_s[ds]
        cur_s = es_s[ds]
        cur_q = eq_s[ds]
        for d in (1, 2, 4, 8):
            idxs = jnp.maximum(iota - d, 0)
            sh_s = cur_s.at[idxs].get(mode="promise_in_bounds")
            sh_q = cur_q.at[idxs].get(mode="promise_in_bounds")
            sh_id = ids16.at[idxs].get(mode="promise_in_bounds")
            same = sh_id == ids16
            cur_q = jnp.where(same & (sh_s >= cur_s), sh_q, cur_q)
            cur_s = jnp.where(same, jnp.maximum(sh_s, cur_s), cur_s)
        nxt_id = ids16.at[nxt_idx].get(mode="promise_in_bounds")
        is_last = (ids16 != nxt_id) | (iota == L - 1)
        mv = plsc.load_gather(mv_s, [ids16])
        upd = is_last & (cur_s > mv)
        plsc.store_scatter(mv_s, [ids16], cur_s, mask=upd)
        plsc.store_scatter(mq_s, [ids16], cur_q, mask=upd)
        return 0

    lax.fori_loop(0, 2 * NV // L, bodyb, 0)

    pltpu.async_copy(mv_s, bestv.at[wid], sem2)
    pltpu.async_copy(mq_s, bestq.at[wid], sem2)
    pltpu.make_async_copy(mv_s, bestv.at[wid], sem2).wait()
    pltpu.make_async_copy(mq_s, bestq.at[wid], sem2).wait()


def _e3_body(bestv_ref, bestq_ref, zpart_ref, cpart_ref, stop_ref, out_ref):
    best = bestv_ref[0]
    q = bestq_ref[0]
    for w in range(1, NW):
        v = bestv_ref[w]
        upd = v > best
        best = jnp.where(upd, v, best)
        q = jnp.where(upd, bestq_ref[w], q)
    z = zpart_ref[0] + zpart_ref[1]
    cnt = cpart_ref[0] + cpart_ref[1]
    exp_stop = jnp.exp(stop_ref[...])
    dinv = 1.0 / (z + exp_stop)
    it = 1.0 / (cnt + 1.0)
    final_stop = (1.0 - RAP) * exp_stop * dinv + RAP * it
    log_stop = jnp.log(jnp.clip(final_stop, PROB_EPS, None))
    log_edge = jnp.log(jnp.clip(q, PROB_EPS, None))
    out_ref[...] = jnp.where(final_stop >= best, log_stop, log_edge)


def _e3(bestv, bestq, zpart, cpart, stop_logits):
    rb = B // 128
    out = pl.pallas_call(
        _e3_body,
        out_shape=jax.ShapeDtypeStruct((rb, 128), jnp.float32),
    )(bestv.reshape(NW, rb, 128), bestq.reshape(NW, rb, 128),
      zpart.reshape(NC, rb, 128), cpart.reshape(NC, rb, 128),
      stop_logits.reshape(rb, 128))
    return out.reshape(B)


def kernel(edge_logits, stop_logits, edge_scores, u, edge_batch):
    p2 = _e1a(edge_logits, edge_scores)
    zpart, cpart = _s1(edge_batch, p2.reshape(E))
    w2 = _e1b(u)
    bestv, bestq = _s2(edge_batch, p2.reshape(E), w2.reshape(E),
                       zpart, cpart, stop_logits)
    return _e3(bestv, bestq, zpart, cpart, stop_logits)
